# Initial kernel scaffold; baseline (speedup 1.0000x reference)
import jax
import jax.numpy as jnp
from jax.experimental import pallas as pl


def kernel(x, perm):
    raise NotImplementedError("write your pallas kernel here")


# speedup vs baseline: 1.1131x; 1.1131x over previous
import functools

import jax
import jax.numpy as jnp
from jax import lax
from jax.experimental import pallas as pl
from jax.experimental.pallas import tpu as pltpu
from jax.experimental.pallas import tpu_sc as plsc

M = 500000
K = 250000
NC = 2
NS = 16
NW = NC * NS

P = ((M + NW - 1) // NW + 127) // 128 * 128
assert NW * P >= M and P % 8 == 0 and (M - P) % 8 == 0


@functools.partial(
    pl.kernel,
    out_type=(
        jax.ShapeDtypeStruct((M,), jnp.float32),
        jax.ShapeDtypeStruct((M,), jnp.float32),
        jax.ShapeDtypeStruct((M,), jnp.float32),
    ),
    mesh=plsc.VectorSubcoreMesh(
        core_axis_name="c", subcore_axis_name="s", num_cores=NC, num_subcores=NS
    ),
    scratch_types=[
        pltpu.VMEM((P,), jnp.int32),
        pltpu.VMEM((P,), jnp.float32),
        pltpu.VMEM((P,), jnp.float32),
        pltpu.SemaphoreType.DMA,
        pltpu.SemaphoreType.DMA,
        pltpu.SemaphoreType.DMA,
        pltpu.SemaphoreType.DMA,
    ],
)
def _edge_mask_sc(perm_ref, probs_ref, soft_ref, hard_ref,
                  idx_v, val_v, half_v, sem_i, sem_p, sem_h, sem_s):
    w = lax.axis_index("s") * NC + lax.axis_index("c")
    base = jnp.minimum(w * P, M - P)

    load = pltpu.async_copy(perm_ref.at[pl.ds(base, P)], idx_v, sem_i)

    iota = lax.broadcasted_iota(jnp.int32, (16,), 0)
    one = jnp.full((16,), 1.0, jnp.float32)
    zero = jnp.full((16,), 0.0, jnp.float32)
    half = jnp.full((16,), 0.5, jnp.float32)

    def fill(i, _):
        pos = base + i * 16 + iota
        val_v[pl.ds(i * 16, 16)] = jnp.where(pos < K, one, zero)
        half_v[pl.ds(i * 16, 16)] = half
        return 0

    lax.fori_loop(0, P // 16, fill, 0, unroll=4)

    probs_copy = pltpu.async_copy(half_v, probs_ref.at[pl.ds(base, P)], sem_p)
    load.wait()
    hard_copy = pltpu.async_copy(val_v, hard_ref.at[idx_v], sem_h)
    soft_copy = pltpu.async_copy(val_v, soft_ref.at[idx_v], sem_s)
    probs_copy.wait()
    hard_copy.wait()
    soft_copy.wait()


def kernel(x, perm):
    del x
    probs, soft, hard = _edge_mask_sc(perm.astype(jnp.int32))
    return probs, soft, hard


# speedup vs baseline: 2.0841x; 1.8723x over previous
import functools

import jax
import jax.numpy as jnp
from jax import lax
from jax.experimental import pallas as pl
from jax.experimental.pallas import tpu as pltpu
from jax.experimental.pallas import tpu_sc as plsc

M = 500000
K = 250000
NC = 2
NS = 16
NW = NC * NS

P = ((M + NW - 1) // NW + 127) // 128 * 128
assert NW * P >= M and P % 8 == 0 and (M - P) % 8 == 0


@functools.partial(
    pl.kernel,
    out_type=(
        jax.ShapeDtypeStruct((M,), jnp.float32),
        jax.ShapeDtypeStruct((M,), jnp.float32),
    ),
    mesh=plsc.VectorSubcoreMesh(
        core_axis_name="c", subcore_axis_name="s", num_cores=NC, num_subcores=NS
    ),
    scratch_types=[
        pltpu.VMEM((P,), jnp.int32),
        pltpu.VMEM((P,), jnp.float32),
        pltpu.VMEM((P,), jnp.float32),
        pltpu.SemaphoreType.DMA,
        pltpu.SemaphoreType.DMA,
        pltpu.SemaphoreType.DMA,
    ],
)
def _edge_mask_sc(perm_ref, probs_ref, hard_ref,
                  idx_v, val_v, half_v, sem_i, sem_p, sem_h):
    w = lax.axis_index("s") * NC + lax.axis_index("c")
    base = jnp.minimum(w * P, M - P)

    load = pltpu.async_copy(perm_ref.at[pl.ds(base, P)], idx_v, sem_i)

    iota = lax.broadcasted_iota(jnp.int32, (16,), 0)
    one = jnp.full((16,), 1.0, jnp.float32)
    zero = jnp.full((16,), 0.0, jnp.float32)
    half = jnp.full((16,), 0.5, jnp.float32)

    def fill(i, _):
        pos = base + i * 16 + iota
        val_v[pl.ds(i * 16, 16)] = jnp.where(pos < K, one, zero)
        half_v[pl.ds(i * 16, 16)] = half
        return 0

    lax.fori_loop(0, P // 16, fill, 0, unroll=4)

    probs_copy = pltpu.async_copy(half_v, probs_ref.at[pl.ds(base, P)], sem_p)
    load.wait()
    hard_copy = pltpu.async_copy(val_v, hard_ref.at[idx_v], sem_h)
    probs_copy.wait()
    hard_copy.wait()


def kernel(x, perm):
    del x
    probs, hard = _edge_mask_sc(perm.astype(jnp.int32))
    return probs, hard, hard


# speedup vs baseline: 31.7174x; 15.2188x over previous
import functools

import jax
import jax.numpy as jnp
from jax import lax
from jax.experimental import pallas as pl
from jax.experimental.pallas import tpu as pltpu
from jax.experimental.pallas import tpu_sc as plsc

M = 500000
K = 250000
NC = 2
NS = 16
NW = NC * NS
HALF = M // NC

P16 = ((M + NS - 1) // NS + 127) // 128 * 128
PH = ((HALF + NS - 1) // NS + 7) // 8 * 8
PP = ((M + NW - 1) // NW + 7) // 8 * 8
assert NS * P16 >= M and (M - P16) % 8 == 0
assert NS * PH >= HALF and (HALF - PH) % 8 == 0
assert NW * PP >= M and (M - PP) % 8 == 0


@functools.partial(
    pl.kernel,
    out_type=(
        jax.ShapeDtypeStruct((M,), jnp.float32),
        jax.ShapeDtypeStruct((M,), jnp.float32),
    ),
    mesh=plsc.VectorSubcoreMesh(
        core_axis_name="c", subcore_axis_name="s", num_cores=NC, num_subcores=NS
    ),
    scratch_types=[
        pltpu.VMEM_SHARED((M,), jnp.float32),
        pltpu.VMEM((P16,), jnp.int32),
        pltpu.VMEM((P16,), jnp.float32),
        pltpu.VMEM((PP,), jnp.float32),
        pltpu.SemaphoreType.DMA,
        pltpu.SemaphoreType.DMA,
        pltpu.SemaphoreType.DMA,
        pltpu.SemaphoreType.DMA,
    ],
)
def _edge_mask_sc(perm_ref, probs_ref, hard_ref, mask_sh,
                  idx_v, val_v, half_v, sem_i, sem_s, sem_p, sem_h):
    c = lax.axis_index("c")
    s = lax.axis_index("s")
    sbase = jnp.minimum(s * P16, M - P16)

    load = pltpu.async_copy(perm_ref.at[pl.ds(sbase, P16)], idx_v, sem_i)

    iota = lax.broadcasted_iota(jnp.int32, (16,), 0)
    one = jnp.full((16,), 1.0, jnp.float32)
    zero = jnp.full((16,), 0.0, jnp.float32)
    half = jnp.full((16,), 0.5, jnp.float32)

    def fill_val(i, _):
        pos = sbase + i * 16 + iota
        val_v[pl.ds(i * 16, 16)] = jnp.where(pos < K, one, zero)
        return 0

    lax.fori_loop(0, P16 // 16, fill_val, 0, unroll=8)

    def fill_half(i, _):
        half_v[pl.ds(i * 16, 16)] = half
        return 0

    lax.fori_loop(0, PP // 16, fill_half, 0, unroll=8)

    w = s * NC + c
    pbase = jnp.minimum(w * PP, M - PP)
    probs_copy = pltpu.async_copy(half_v, probs_ref.at[pl.ds(pbase, PP)], sem_p)

    load.wait()
    pltpu.async_copy(val_v, mask_sh.at[idx_v], sem_s).wait()
    plsc.subcore_barrier()

    hbase = c * HALF + jnp.minimum(s * PH, HALF - PH)
    pltpu.sync_copy(mask_sh.at[pl.ds(hbase, PH)], val_v.at[pl.ds(0, PH)])
    out_copy = pltpu.async_copy(
        val_v.at[pl.ds(0, PH)], hard_ref.at[pl.ds(hbase, PH)], sem_h
    )
    probs_copy.wait()
    out_copy.wait()


def kernel(x, perm):
    del x
    probs, hard = _edge_mask_sc(perm.astype(jnp.int32))
    return probs, hard, hard


# speedup vs baseline: 34.0529x; 1.0736x over previous
import functools

import jax
import jax.numpy as jnp
from jax import lax
from jax.experimental import pallas as pl
from jax.experimental.pallas import tpu as pltpu
from jax.experimental.pallas import tpu_sc as plsc

M = 500000
K = 250000
NC = 2
NS = 16
NW = NC * NS
HALF = M // NC

P16 = ((M + NS - 1) // NS + 127) // 128 * 128
PH = ((HALF + NS - 1) // NS + 7) // 8 * 8
PP = ((M + NW - 1) // NW + 7) // 8 * 8
assert NS * P16 >= M and (M - P16) % 8 == 0
assert NS * PH >= HALF and (HALF - PH) % 8 == 0
assert NW * PP >= M and (M - PP) % 8 == 0


@functools.partial(
    pl.kernel,
    out_type=(
        jax.ShapeDtypeStruct((M,), jnp.float32),
        jax.ShapeDtypeStruct((M,), jnp.float32),
        jax.ShapeDtypeStruct((M,), jnp.float32),
    ),
    mesh=plsc.VectorSubcoreMesh(
        core_axis_name="c", subcore_axis_name="s", num_cores=NC, num_subcores=NS
    ),
    scratch_types=[
        pltpu.VMEM_SHARED((M,), jnp.float32),
        pltpu.VMEM((P16,), jnp.int32),
        pltpu.VMEM((P16,), jnp.float32),
        pltpu.VMEM((PP,), jnp.float32),
        pltpu.SemaphoreType.DMA,
        pltpu.SemaphoreType.DMA,
        pltpu.SemaphoreType.DMA,
        pltpu.SemaphoreType.DMA,
        pltpu.SemaphoreType.DMA,
    ],
)
def _edge_mask_sc(perm_ref, probs_ref, soft_ref, hard_ref, mask_sh,
                  idx_v, val_v, half_v, sem_i, sem_s, sem_p, sem_h, sem_h2):
    c = lax.axis_index("c")
    s = lax.axis_index("s")
    sbase = jnp.minimum(s * P16, M - P16)

    load = pltpu.async_copy(perm_ref.at[pl.ds(sbase, P16)], idx_v, sem_i)

    iota = lax.broadcasted_iota(jnp.int32, (16,), 0)
    one = jnp.full((16,), 1.0, jnp.float32)
    zero = jnp.full((16,), 0.0, jnp.float32)
    half = jnp.full((16,), 0.5, jnp.float32)

    with jax.named_scope("fill"):
        def fill_val(i, _):
            pos = sbase + i * 16 + iota
            val_v[pl.ds(i * 16, 16)] = jnp.where(pos < K, one, zero)
            return 0

        lax.fori_loop(0, P16 // 16, fill_val, 0, unroll=8)

        def fill_half(i, _):
            half_v[pl.ds(i * 16, 16)] = half
            return 0

        lax.fori_loop(0, PP // 16, fill_half, 0, unroll=8)

    w = s * NC + c
    pbase = jnp.minimum(w * PP, M - PP)
    probs_copy = pltpu.async_copy(half_v, probs_ref.at[pl.ds(pbase, PP)], sem_p)

    with jax.named_scope("scatter"):
        load.wait()
        pltpu.async_copy(val_v, mask_sh.at[idx_v], sem_s).wait()
    plsc.subcore_barrier()

    with jax.named_scope("writeout"):
        hbase = c * HALF + jnp.minimum(s * PH, HALF - PH)
        pltpu.sync_copy(mask_sh.at[pl.ds(hbase, PH)], val_v.at[pl.ds(0, PH)])
        out_copy = pltpu.async_copy(
            val_v.at[pl.ds(0, PH)], hard_ref.at[pl.ds(hbase, PH)], sem_h
        )
        out_copy2 = pltpu.async_copy(
            val_v.at[pl.ds(0, PH)], soft_ref.at[pl.ds(hbase, PH)], sem_h2
        )
        probs_copy.wait()
        out_copy.wait()
        out_copy2.wait()


def kernel(x, perm):
    del x
    probs, soft, hard = _edge_mask_sc(perm.astype(jnp.int32))
    return probs, soft, hard


# speedup vs baseline: 38.4675x; 1.1296x over previous
import functools

import jax
import jax.numpy as jnp
from jax import lax
from jax.experimental import pallas as pl
from jax.experimental.pallas import tpu as pltpu
from jax.experimental.pallas import tpu_sc as plsc

M = 500000
K = 250000
NC = 2
NS = 16
NW = NC * NS
HALF = M // NC

PZ = ((M + NS - 1) // NS + 127) // 128 * 128
PK = ((K + NS - 1) // NS + 127) // 128 * 128
PH = ((HALF + NS - 1) // NS + 7) // 8 * 8
PP = ((M + NW - 1) // NW + 7) // 8 * 8
assert NS * PZ >= M and (M - PZ) % 8 == 0
assert NS * PK >= K and (K - PK) % 8 == 0
assert NS * PH >= HALF and (HALF - PH) % 8 == 0
assert NW * PP >= M and (M - PP) % 8 == 0


@functools.partial(
    pl.kernel,
    out_type=(
        jax.ShapeDtypeStruct((M,), jnp.float32),
        jax.ShapeDtypeStruct((M,), jnp.float32),
        jax.ShapeDtypeStruct((M,), jnp.float32),
    ),
    mesh=plsc.VectorSubcoreMesh(
        core_axis_name="c", subcore_axis_name="s", num_cores=NC, num_subcores=NS
    ),
    scratch_types=[
        pltpu.VMEM_SHARED((M,), jnp.float32),
        pltpu.VMEM((PK,), jnp.int32),
        pltpu.VMEM((PZ,), jnp.float32),
        pltpu.VMEM((PP,), jnp.float32),
        pltpu.SemaphoreType.DMA,
        pltpu.SemaphoreType.DMA,
        pltpu.SemaphoreType.DMA,
        pltpu.SemaphoreType.DMA,
        pltpu.SemaphoreType.DMA,
        pltpu.SemaphoreType.DMA,
    ],
)
def _edge_mask_sc(perm_ref, probs_ref, soft_ref, hard_ref, mask_sh,
                  idx_v, val_v, half_v,
                  sem_i, sem_z, sem_s, sem_p, sem_h, sem_h2):
    c = lax.axis_index("c")
    s = lax.axis_index("s")

    kbase = jnp.minimum(s * PK, K - PK)
    load = pltpu.async_copy(perm_ref.at[pl.ds(kbase, PK)], idx_v, sem_i)

    one = jnp.full((16,), 1.0, jnp.float32)
    zero = jnp.full((16,), 0.0, jnp.float32)
    half = jnp.full((16,), 0.5, jnp.float32)

    def fill_zero(i, _):
        val_v[pl.ds(i * 16, 16)] = zero
        return 0

    lax.fori_loop(0, PZ // 16, fill_zero, 0, unroll=8)

    zbase = jnp.minimum(s * PZ, M - PZ)
    zcopy = pltpu.async_copy(val_v, mask_sh.at[pl.ds(zbase, PZ)], sem_z)

    def fill_half(i, _):
        half_v[pl.ds(i * 16, 16)] = half
        return 0

    lax.fori_loop(0, PP // 16, fill_half, 0, unroll=8)

    w = s * NC + c
    pbase = jnp.minimum(w * PP, M - PP)
    probs_copy = pltpu.async_copy(half_v, probs_ref.at[pl.ds(pbase, PP)], sem_p)

    zcopy.wait()

    def fill_one(i, _):
        val_v[pl.ds(i * 16, 16)] = one
        return 0

    lax.fori_loop(0, PK // 16, fill_one, 0, unroll=8)

    plsc.subcore_barrier()

    load.wait()
    pltpu.async_copy(val_v.at[pl.ds(0, PK)], mask_sh.at[idx_v], sem_s).wait()
    plsc.subcore_barrier()

    hbase = c * HALF + jnp.minimum(s * PH, HALF - PH)
    pltpu.sync_copy(mask_sh.at[pl.ds(hbase, PH)], val_v.at[pl.ds(0, PH)])
    out_h = pltpu.async_copy(
        val_v.at[pl.ds(0, PH)], hard_ref.at[pl.ds(hbase, PH)], sem_h
    )
    out_s = pltpu.async_copy(
        val_v.at[pl.ds(0, PH)], soft_ref.at[pl.ds(hbase, PH)], sem_h2
    )
    probs_copy.wait()
    out_h.wait()
    out_s.wait()


def kernel(x, perm):
    del x
    probs, soft, hard = _edge_mask_sc(perm.astype(jnp.int32))
    return probs, soft, hard


# speedup vs baseline: 39.4108x; 1.0245x over previous
import functools

import jax
import jax.numpy as jnp
from jax import lax
from jax.experimental import pallas as pl
from jax.experimental.pallas import tpu as pltpu
from jax.experimental.pallas import tpu_sc as plsc

M = 500000
K = 250000
NC = 2
NS = 16
NW = NC * NS
HALF = M // NC

PZ = ((M + NS - 1) // NS + 127) // 128 * 128
PK = ((K + NS - 1) // NS + 127) // 128 * 128
PH = ((HALF + NS - 1) // NS + 7) // 8 * 8
PP = ((M + NW - 1) // NW + 15) // 16 * 16
REP = 245
assert NS * PZ >= M and (M - PZ) % 8 == 0
assert NS * PK >= K and (K - PK) % 8 == 0
assert NS * PH >= HALF and (HALF - PH) % 8 == 0
assert NW * PP >= M and (M - PP) % 8 == 0
assert PZ % (16 * REP) == 0 and PP % 8 == 0
NREP_Z = PZ // (16 * REP)
assert PP % (16 * REP) == 0 or PP < 16 * REP or True


@functools.partial(
    pl.kernel,
    out_type=(
        jax.ShapeDtypeStruct((M,), jnp.float32),
        jax.ShapeDtypeStruct((M,), jnp.float32),
        jax.ShapeDtypeStruct((M,), jnp.float32),
    ),
    mesh=plsc.VectorSubcoreMesh(
        core_axis_name="c", subcore_axis_name="s", num_cores=NC, num_subcores=NS
    ),
    scratch_types=[
        pltpu.VMEM_SHARED((M,), jnp.float32),
        pltpu.VMEM((PK,), jnp.int32),
        pltpu.VMEM((PK,), jnp.float32),
        pltpu.VMEM((16 * REP,), jnp.float32),
        pltpu.VMEM((16 * REP,), jnp.float32),
        pltpu.VMEM((PH,), jnp.float32),
        pltpu.SemaphoreType.DMA,
        pltpu.SemaphoreType.DMA,
        pltpu.SemaphoreType.DMA,
        pltpu.SemaphoreType.DMA,
        pltpu.SemaphoreType.DMA,
        pltpu.SemaphoreType.DMA,
    ],
)
def _edge_mask_sc(perm_ref, probs_ref, soft_ref, hard_ref, mask_sh,
                  idx_v, one_v, zero_v, half_v, out_v,
                  sem_i, sem_z, sem_s, sem_p, sem_h, sem_h2):
    c = lax.axis_index("c")
    s = lax.axis_index("s")

    kbase = jnp.minimum(s * PK, K - PK)
    load = pltpu.async_copy(perm_ref.at[pl.ds(kbase, PK)], idx_v, sem_i)

    one = jnp.full((16,), 1.0, jnp.float32)
    zero = jnp.full((16,), 0.0, jnp.float32)
    half = jnp.full((16,), 0.5, jnp.float32)

    def fill_zero(i, _):
        zero_v[pl.ds(i * 16, 16)] = zero
        return 0

    lax.fori_loop(0, REP, fill_zero, 0, unroll=8)

    zbase = jnp.minimum(s * PZ, M - PZ)
    zcopies = [
        pltpu.async_copy(
            zero_v, mask_sh.at[pl.ds(zbase + j * 16 * REP, 16 * REP)], sem_z
        )
        for j in range(NREP_Z)
    ]

    def fill_half(i, _):
        half_v[pl.ds(i * 16, 16)] = half
        return 0

    lax.fori_loop(0, REP, fill_half, 0, unroll=8)

    def fill_one(i, _):
        one_v[pl.ds(i * 16, 16)] = one
        return 0

    lax.fori_loop(0, PK // 16, fill_one, 0, unroll=8)

    w = s * NC + c
    pbase = jnp.minimum(w * PP, M - PP)
    pcopies = []
    off = 0
    while off < PP:
        n = min(16 * REP, PP - off)
        pcopies.append(
            pltpu.async_copy(
                half_v.at[pl.ds(0, n)], probs_ref.at[pl.ds(pbase + off, n)],
                sem_p,
            )
        )
        off += n

    for zc in zcopies:
        zc.wait()
    plsc.subcore_barrier()
    plsc.subcore_barrier()

    load.wait()
    scat = pltpu.async_copy(one_v, mask_sh.at[idx_v], sem_s)
    scat.wait()
    plsc.subcore_barrier()
    plsc.subcore_barrier()

    hbase = c * HALF + jnp.minimum(s * PH, HALF - PH)
    pltpu.sync_copy(mask_sh.at[pl.ds(hbase, PH)], out_v)
    out_h = pltpu.async_copy(out_v, hard_ref.at[pl.ds(hbase, PH)], sem_h)
    out_s = pltpu.async_copy(out_v, soft_ref.at[pl.ds(hbase, PH)], sem_h2)
    for cp in pcopies:
        cp.wait()
    out_h.wait()
    out_s.wait()


def kernel(x, perm):
    del x
    probs, soft, hard = _edge_mask_sc(perm.astype(jnp.int32))
    return probs, soft, hard
